# baseline (speedup 1.0000x reference)
import functools

import jax
import jax.numpy as jnp
from jax import lax
from jax.experimental import pallas as pl
from jax.experimental.pallas import tpu as pltpu
from jax.experimental.pallas import tpu_sc as plsc

N_NODES = 10000
N_EDGES = 160000
D = 256
HID = 64

BM = 1000
_NEG_INF = float("-inf")

NW = 32
NPW = 320
N_PAD = NW * NPW
CHUNK = 3200
SG = 64
NSG = CHUNK // SG
NCHUNK = N_EDGES // CHUNK
G = 128


def _leaky(x):
    return jnp.where(x >= 0, x, 0.01 * x)


def _mlp3(z, w1, b1, w2, b2, w3, b3):
    h = _leaky(jnp.dot(z, w1, preferred_element_type=jnp.float32) + b1)
    h = _leaky(jnp.dot(h, w2, preferred_element_type=jnp.float32) + b2)
    return jnp.dot(h, w3, preferred_element_type=jnp.float32) + b3


def _gin_apply_kernel(x_ref, agg_ref, w1, b1, w2, b2, w3, b3, o_ref):
    z = x_ref[...] + agg_ref[...]
    o = _mlp3(z, w1[...], b1[...], w2[...], b2[...], w3[...], b3[...])
    o_ref[...] = jnp.maximum(o, 0.0)


def _gin_apply(x, agg, w1, b1, w2, b2, w3, b3):
    grid = (N_NODES // BM,)
    row = pl.BlockSpec((BM, D), lambda i: (i, 0))
    full = lambda a: pl.BlockSpec(a.shape, lambda i: (0,) * a.ndim)
    return pl.pallas_call(
        _gin_apply_kernel,
        grid=grid,
        in_specs=[row, row, full(w1), full(b1), full(w2), full(b2),
                  full(w3), full(b3)],
        out_specs=row,
        out_shape=jax.ShapeDtypeStruct((N_NODES, D), jnp.float32),
    )(x, agg, w1, b1, w2, b2, w3, b3)


def _final_kernel(a_ref, agg_ref, w1, b1, w2, b2, w3, b3,
                  g1, gb1, g2, gb2, g3, gb3, rs_ref, o_ref,
                  m_s, s_s, v_s):
    i = pl.program_id(0)

    @pl.when(i == 0)
    def _():
        m_s[0, 0] = _NEG_INF
        s_s[0, 0] = 0.0
        v_s[...] = jnp.zeros_like(v_s)

    a = a_ref[...]
    z = a + agg_ref[...]
    b = jnp.maximum(_mlp3(z, w1[...], b1[...], w2[...], b2[...],
                          w3[...], b3[...]), 0.0)
    h = a + rs_ref[0, 0] * b
    g = _mlp3(h, g1[...], gb1[...], g2[...], gb2[...], g3[...], gb3[...])

    m_old = m_s[0, 0]
    m_new = jnp.maximum(m_old, jnp.max(g))
    c = jnp.exp(m_old - m_new)
    w = jnp.exp(g - m_new)
    s_s[0, 0] = s_s[0, 0] * c + jnp.sum(w)
    wv = lax.dot_general(w, h, (((0,), (0,)), ((), ())),
                         preferred_element_type=jnp.float32)
    v_s[...] = v_s[...] * c + wv
    m_s[0, 0] = m_new

    @pl.when(i == pl.num_programs(0) - 1)
    def _():
        o_ref[...] = v_s[...] / s_s[0, 0]


def _final_stage(a, agg, w1, b1, w2, b2, w3, b3, g1, gb1, g2, gb2, g3, gb3, rs):
    grid = (N_NODES // BM,)
    row = pl.BlockSpec((BM, D), lambda i: (i, 0))
    full = lambda x: pl.BlockSpec(x.shape, lambda i: (0,) * x.ndim)
    return pl.pallas_call(
        _final_kernel,
        grid=grid,
        in_specs=[row, row, full(w1), full(b1), full(w2), full(b2),
                  full(w3), full(b3), full(g1), full(gb1), full(g2),
                  full(gb2), full(g3), full(gb3), full(rs)],
        out_specs=pl.BlockSpec((1, D), lambda i: (0, 0)),
        out_shape=jax.ShapeDtypeStruct((1, D), jnp.float32),
        scratch_shapes=[
            pltpu.SMEM((1, 1), jnp.float32),
            pltpu.SMEM((1, 1), jnp.float32),
            pltpu.VMEM((1, D), jnp.float32),
        ],
    )(a, agg, w1, b1, w2, b2, w3, b3, g1, gb1, g2, gb2, g3, gb3, rs)


def _make_sc_segmax(zero_init):
    init_val = 0.0 if zero_init else _NEG_INF
    mesh = plsc.VectorSubcoreMesh(core_axis_name="c", subcore_axis_name="s",
                                  num_cores=2, num_subcores=16)

    @functools.partial(
        pl.kernel,
        out_type=jax.ShapeDtypeStruct((N_PAD, D), jnp.float32),
        mesh=mesh,
        scratch_types=[
            pltpu.VMEM((NPW + 1, D), jnp.float32),
            pltpu.VMEM((CHUNK,), jnp.int32),
            pltpu.VMEM((CHUNK,), jnp.int32),
            pltpu.VMEM((G,), jnp.int32),
            pltpu.VMEM((G + 16,), jnp.int32),
            pltpu.VMEM((G, D), jnp.float32),
            pltpu.SemaphoreType.DMA,
        ],
    )
    def seg(x_hbm, src_hbm, dst_hbm, out_hbm, acc, srcb, dstb, seli, seld,
            msg, sem):
        wid = lax.axis_index("s") * 2 + lax.axis_index("c")
        base = wid * NPW

        def init_row(r, carry):
            for k in range(D // 16):
                acc[r, pl.ds(16 * k, 16)] = jnp.full((16,), init_val,
                                                     jnp.float32)
            return carry
        lax.fori_loop(0, NPW + 1, init_row, 0)
        npw_vec = jnp.full((16,), NPW, jnp.int32)
        for k in range(G // 16):
            seli[pl.ds(16 * k, 16)] = jnp.zeros((16,), jnp.int32)
        for k in range(G // 16 + 1):
            seld[pl.ds(16 * k, 16)] = npw_vec

        def flush(cu):
            seld[pl.ds(cu, 16)] = npw_vec
            pltpu.async_copy(x_hbm.at[seli], msg, sem).wait()

            def upd(jb, carry):
                jo = pl.multiple_of(jb * 16, 8)
                ldv = seld[pl.ds(jo, 16)]
                for jj in range(16):
                    r = ldv[jj]
                    for k in range(D // 16):
                        s = pl.ds(16 * k, 16)
                        acc[r, s] = jnp.maximum(
                            acc[r, s], msg[jb * 16 + jj, s])
                return carry
            lax.fori_loop(0, G // 16, upd, 0)
            for k in range(G // 16 + 1):
                seld[pl.ds(16 * k, 16)] = npw_vec

        lanes = lax.iota(jnp.int32, 16)
        ones = jnp.full((16,), 1, jnp.int32)
        zeros = jnp.zeros((16,), jnp.int32)

        def chunk_body(c, cursor):
            off = pl.multiple_of(c * CHUNK, 8)
            pltpu.sync_copy(src_hbm.at[pl.ds(off, CHUNK)], srcb)
            pltpu.sync_copy(dst_hbm.at[pl.ds(off, CHUNK)], dstb)

            def sg_body(gsg, cu):
                o = gsg * SG
                lds, svs, prefs, cnts = [], [], [], []
                for k in range(SG // 16):
                    ok = pl.multiple_of(o + 16 * k, 8)
                    ld = dstb[pl.ds(ok, 16)] - base
                    sv = srcb[pl.ds(ok, 16)]
                    m = (ld >= 0) & (ld < NPW)
                    s = jnp.where(m, ones, zeros)
                    for sh in (1, 2, 4, 8):
                        sg_ = s[jnp.maximum(lanes - sh, 0)]
                        s = s + jnp.where(lanes >= sh, sg_, zeros)
                    lds.append(ld)
                    svs.append(sv)
                    prefs.append(s)
                    cnts.append(s[15])

                def do_flush(c0):
                    flush(c0)
                    return 0
                cu = lax.cond(cu > G - SG, do_flush, lambda c0: c0, cu)

                for k in range(SG // 16):
                    s = prefs[k]
                    tgt = lanes + 1
                    inv = zeros
                    for step in (8, 4, 2, 1):
                        probe = inv + (step - 1)
                        v = s[jnp.minimum(probe, 15)]
                        inv = jnp.where(v < tgt, inv + step, inv)
                    seli[pl.ds(cu, 16)] = svs[k][inv]
                    seld[pl.ds(cu, 16)] = lds[k][inv]
                    cu = cu + cnts[k]
                return cu

            return lax.fori_loop(0, NSG, sg_body, cursor)

        cursor = lax.fori_loop(0, NCHUNK, chunk_body, 0)
        flush(cursor)

        if not zero_init:
            def fix_row(r, carry):
                for k in range(D // 16):
                    s = pl.ds(16 * k, 16)
                    v = acc[r, s]
                    acc[r, s] = jnp.where(v == _NEG_INF, 0.0, v)
                return carry
            lax.fori_loop(0, NPW, fix_row, 0)

        pltpu.sync_copy(acc.at[pl.ds(0, NPW)], out_hbm.at[pl.ds(base, NPW)])

    return seg


_segmax_neg = _make_sc_segmax(zero_init=False)
_segmax_zero = _make_sc_segmax(zero_init=True)


def _segmax(x, src, dst, zero_init):
    fn = _segmax_zero if zero_init else _segmax_neg
    return fn(x, src, dst)[:N_NODES]


def kernel(node_feat, edge_index, W1, b1, W2, b2, W3, b3,
           G1, gb1, G2, gb2, G3, gb3, residual_scale):
    src = edge_index[0].astype(jnp.int32)
    dst = edge_index[1].astype(jnp.int32)
    b1r = b1.reshape(1, HID)
    b2r = b2.reshape(1, HID)
    b3r = b3.reshape(1, D)
    gb1r = gb1.reshape(1, HID)
    gb2r = gb2.reshape(1, HID)
    gb3r = gb3.reshape(1, 1)
    rs = residual_scale.reshape(1, 1)

    agg_x = _segmax(node_feat, src, dst, zero_init=False)
    a = _gin_apply(node_feat, agg_x, W1, b1r, W2, b2r, W3, b3r)
    agg_a = _segmax(a, src, dst, zero_init=True)
    return _final_stage(a, agg_a, W1, b1r, W2, b2r, W3, b3r,
                        G1, gb1r, G2, gb2r, G3, gb3r, rs)


# speedup vs baseline: 3.2369x; 3.2369x over previous
import functools

import jax
import jax.numpy as jnp
from jax import lax
from jax.experimental import pallas as pl
from jax.experimental.pallas import tpu as pltpu
from jax.experimental.pallas import tpu_sc as plsc

N_NODES = 10000
N_EDGES = 160000
D = 256
HID = 64

BM = 1000
_NEG_INF = float("-inf")

NW = 32
NPW = 320
N_PAD = NW * NPW
CHUNK = 3200
SG = 64
NSG = CHUNK // SG
NCHUNK = N_EDGES // CHUNK
G = 128


def _leaky(x):
    return jnp.where(x >= 0, x, 0.01 * x)


def _mlp3(z, w1, b1, w2, b2, w3, b3):
    h = _leaky(jnp.dot(z, w1, preferred_element_type=jnp.float32) + b1)
    h = _leaky(jnp.dot(h, w2, preferred_element_type=jnp.float32) + b2)
    return jnp.dot(h, w3, preferred_element_type=jnp.float32) + b3


def _gin_apply_kernel(x_ref, agg_ref, w1, b1, w2, b2, w3, b3, o_ref):
    z = x_ref[...] + agg_ref[...]
    o = _mlp3(z, w1[...], b1[...], w2[...], b2[...], w3[...], b3[...])
    o_ref[...] = jnp.maximum(o, 0.0)


def _gin_apply(x, agg, w1, b1, w2, b2, w3, b3):
    grid = (N_NODES // BM,)
    row = pl.BlockSpec((BM, D), lambda i: (i, 0))
    full = lambda a: pl.BlockSpec(a.shape, lambda i: (0,) * a.ndim)
    return pl.pallas_call(
        _gin_apply_kernel,
        grid=grid,
        in_specs=[row, row, full(w1), full(b1), full(w2), full(b2),
                  full(w3), full(b3)],
        out_specs=row,
        out_shape=jax.ShapeDtypeStruct((N_NODES, D), jnp.float32),
    )(x, agg, w1, b1, w2, b2, w3, b3)


def _final_kernel(a_ref, agg_ref, w1, b1, w2, b2, w3, b3,
                  g1, gb1, g2, gb2, g3, gb3, rs_ref, o_ref,
                  m_s, s_s, v_s):
    i = pl.program_id(0)

    @pl.when(i == 0)
    def _():
        m_s[0, 0] = _NEG_INF
        s_s[0, 0] = 0.0
        v_s[...] = jnp.zeros_like(v_s)

    a = a_ref[...]
    z = a + agg_ref[...]
    b = jnp.maximum(_mlp3(z, w1[...], b1[...], w2[...], b2[...],
                          w3[...], b3[...]), 0.0)
    h = a + rs_ref[0, 0] * b
    g = _mlp3(h, g1[...], gb1[...], g2[...], gb2[...], g3[...], gb3[...])

    m_old = m_s[0, 0]
    m_new = jnp.maximum(m_old, jnp.max(g))
    c = jnp.exp(m_old - m_new)
    w = jnp.exp(g - m_new)
    s_s[0, 0] = s_s[0, 0] * c + jnp.sum(w)
    wv = lax.dot_general(w, h, (((0,), (0,)), ((), ())),
                         preferred_element_type=jnp.float32)
    v_s[...] = v_s[...] * c + wv
    m_s[0, 0] = m_new

    @pl.when(i == pl.num_programs(0) - 1)
    def _():
        o_ref[...] = v_s[...] / s_s[0, 0]


def _final_stage(a, agg, w1, b1, w2, b2, w3, b3, g1, gb1, g2, gb2, g3, gb3, rs):
    grid = (N_NODES // BM,)
    row = pl.BlockSpec((BM, D), lambda i: (i, 0))
    full = lambda x: pl.BlockSpec(x.shape, lambda i: (0,) * x.ndim)
    return pl.pallas_call(
        _final_kernel,
        grid=grid,
        in_specs=[row, row, full(w1), full(b1), full(w2), full(b2),
                  full(w3), full(b3), full(g1), full(gb1), full(g2),
                  full(gb2), full(g3), full(gb3), full(rs)],
        out_specs=pl.BlockSpec((1, D), lambda i: (0, 0)),
        out_shape=jax.ShapeDtypeStruct((1, D), jnp.float32),
        scratch_shapes=[
            pltpu.SMEM((1, 1), jnp.float32),
            pltpu.SMEM((1, 1), jnp.float32),
            pltpu.VMEM((1, D), jnp.float32),
        ],
    )(a, agg, w1, b1, w2, b2, w3, b3, g1, gb1, g2, gb2, g3, gb3, rs)


def _make_sc_segmax(zero_init):
    init_val = 0.0 if zero_init else _NEG_INF
    mesh = plsc.VectorSubcoreMesh(core_axis_name="c", subcore_axis_name="s",
                                  num_cores=2, num_subcores=16)

    @functools.partial(
        pl.kernel,
        out_type=jax.ShapeDtypeStruct((N_PAD, D), jnp.float32),
        mesh=mesh,
        scratch_types=[
            pltpu.VMEM((NPW + 1, D), jnp.float32),
            pltpu.VMEM((CHUNK,), jnp.int32),
            pltpu.VMEM((CHUNK,), jnp.int32),
            pltpu.VMEM((G,), jnp.int32),
            pltpu.VMEM((G + 16,), jnp.int32),
            pltpu.VMEM((G, D), jnp.float32),
            pltpu.SemaphoreType.DMA,
        ],
    )
    def seg(x_hbm, src_hbm, dst_hbm, out_hbm, acc, srcb, dstb, seli, seld,
            msg, sem):
        wid = lax.axis_index("s") * 2 + lax.axis_index("c")
        base = wid * NPW

        def init_row(r, carry):
            for k in range(D // 16):
                acc[r, pl.ds(16 * k, 16)] = jnp.full((16,), init_val,
                                                     jnp.float32)
            return carry
        lax.fori_loop(0, NPW + 1, init_row, 0)
        npw_vec = jnp.full((16,), NPW, jnp.int32)
        slot_iota = lax.iota(jnp.int32, 16)
        for k in range(G // 16):
            seli[pl.ds(16 * k, 16)] = slot_iota + (16 * k)
        for k in range(G // 16 + 1):
            seld[pl.ds(16 * k, 16)] = npw_vec

        def flush(cu):
            seld[pl.ds(cu, 16)] = npw_vec
            seli[pl.ds(cu, 16)] = slot_iota + cu
            pltpu.async_copy(x_hbm.at[seli], msg, sem).wait()

            def upd(jb, carry):
                jo = pl.multiple_of(jb * 16, 8)
                ldv = seld[pl.ds(jo, 16)]
                for jj in range(16):
                    r = ldv[jj]
                    for k in range(D // 16):
                        s = pl.ds(16 * k, 16)
                        acc[r, s] = jnp.maximum(
                            acc[r, s], msg[jb * 16 + jj, s])
                return carry
            lax.fori_loop(0, G // 16, upd, 0)
            for k in range(G // 16 + 1):
                seld[pl.ds(16 * k, 16)] = npw_vec

        lanes = lax.iota(jnp.int32, 16)
        ones = jnp.full((16,), 1, jnp.int32)
        zeros = jnp.zeros((16,), jnp.int32)

        def chunk_body(c, cursor):
            off = pl.multiple_of(c * CHUNK, 8)
            pltpu.sync_copy(src_hbm.at[pl.ds(off, CHUNK)], srcb)
            pltpu.sync_copy(dst_hbm.at[pl.ds(off, CHUNK)], dstb)

            def sg_body(gsg, cu):
                o = gsg * SG
                lds, svs, prefs, cnts = [], [], [], []
                for k in range(SG // 16):
                    ok = pl.multiple_of(o + 16 * k, 8)
                    ld = dstb[pl.ds(ok, 16)] - base
                    sv = srcb[pl.ds(ok, 16)]
                    m = (ld >= 0) & (ld < NPW)
                    s = jnp.where(m, ones, zeros)
                    for sh in (1, 2, 4, 8):
                        sg_ = s[jnp.maximum(lanes - sh, 0)]
                        s = s + jnp.where(lanes >= sh, sg_, zeros)
                    lds.append(ld)
                    svs.append(sv)
                    prefs.append(s)
                    cnts.append(s[15])

                def do_flush(c0):
                    flush(c0)
                    return 0
                cu = lax.cond(cu > G - SG - 16, do_flush, lambda c0: c0, cu)

                for k in range(SG // 16):
                    s = prefs[k]
                    tgt = lanes + 1
                    inv = zeros
                    for step in (8, 4, 2, 1):
                        probe = inv + (step - 1)
                        v = s[jnp.minimum(probe, 15)]
                        inv = jnp.where(v < tgt, inv + step, inv)
                    seli[pl.ds(cu, 16)] = svs[k][inv]
                    seld[pl.ds(cu, 16)] = lds[k][inv]
                    cu = cu + cnts[k]
                return cu

            return lax.fori_loop(0, NSG, sg_body, cursor)

        cursor = lax.fori_loop(0, NCHUNK, chunk_body, 0)
        flush(cursor)

        if not zero_init:
            def fix_row(r, carry):
                for k in range(D // 16):
                    s = pl.ds(16 * k, 16)
                    v = acc[r, s]
                    acc[r, s] = jnp.where(v == _NEG_INF, 0.0, v)
                return carry
            lax.fori_loop(0, NPW, fix_row, 0)

        pltpu.sync_copy(acc.at[pl.ds(0, NPW)], out_hbm.at[pl.ds(base, NPW)])

    return seg


_segmax_neg = _make_sc_segmax(zero_init=False)
_segmax_zero = _make_sc_segmax(zero_init=True)


def _segmax(x, src, dst, zero_init):
    fn = _segmax_zero if zero_init else _segmax_neg
    return fn(x, src, dst)[:N_NODES]


def kernel(node_feat, edge_index, W1, b1, W2, b2, W3, b3,
           G1, gb1, G2, gb2, G3, gb3, residual_scale):
    src = edge_index[0].astype(jnp.int32)
    dst = edge_index[1].astype(jnp.int32)
    b1r = b1.reshape(1, HID)
    b2r = b2.reshape(1, HID)
    b3r = b3.reshape(1, D)
    gb1r = gb1.reshape(1, HID)
    gb2r = gb2.reshape(1, HID)
    gb3r = gb3.reshape(1, 1)
    rs = residual_scale.reshape(1, 1)

    agg_x = _segmax(node_feat, src, dst, zero_init=False)
    a = _gin_apply(node_feat, agg_x, W1, b1r, W2, b2r, W3, b3r)
    agg_a = _segmax(a, src, dst, zero_init=True)
    return _final_stage(a, agg_a, W1, b1r, W2, b2r, W3, b3r,
                        G1, gb1r, G2, gb2r, G3, gb3r, rs)


# speedup vs baseline: 4.8533x; 1.4994x over previous
import functools

import jax
import jax.numpy as jnp
from jax import lax
from jax.experimental import pallas as pl
from jax.experimental.pallas import tpu as pltpu
from jax.experimental.pallas import tpu_sc as plsc

N_NODES = 10000
N_EDGES = 160000
D = 256
HID = 64

BM = 1000
_NEG_INF = float("-inf")

NW = 32
NPW = 320
N_PAD = NW * NPW
CHUNK = 3200
SG = 64
NSG = CHUNK // SG
NCHUNK = N_EDGES // CHUNK
G = 128


def _leaky(x):
    return jnp.where(x >= 0, x, 0.01 * x)


def _mlp3(z, w1, b1, w2, b2, w3, b3):
    h = _leaky(jnp.dot(z, w1, preferred_element_type=jnp.float32) + b1)
    h = _leaky(jnp.dot(h, w2, preferred_element_type=jnp.float32) + b2)
    return jnp.dot(h, w3, preferred_element_type=jnp.float32) + b3


def _gin_apply_kernel(x_ref, agg_ref, w1, b1, w2, b2, w3, b3, o_ref):
    z = x_ref[...] + agg_ref[...]
    o = _mlp3(z, w1[...], b1[...], w2[...], b2[...], w3[...], b3[...])
    o_ref[...] = jnp.maximum(o, 0.0)


def _gin_apply(x, agg, w1, b1, w2, b2, w3, b3):
    grid = (N_NODES // BM,)
    row = pl.BlockSpec((BM, D), lambda i: (i, 0))
    full = lambda a: pl.BlockSpec(a.shape, lambda i: (0,) * a.ndim)
    return pl.pallas_call(
        _gin_apply_kernel,
        grid=grid,
        in_specs=[row, row, full(w1), full(b1), full(w2), full(b2),
                  full(w3), full(b3)],
        out_specs=row,
        out_shape=jax.ShapeDtypeStruct((N_NODES, D), jnp.float32),
    )(x, agg, w1, b1, w2, b2, w3, b3)


def _final_kernel(a_ref, agg_ref, w1, b1, w2, b2, w3, b3,
                  g1, gb1, g2, gb2, g3, gb3, rs_ref, o_ref,
                  m_s, s_s, v_s):
    i = pl.program_id(0)

    @pl.when(i == 0)
    def _():
        m_s[0, 0] = _NEG_INF
        s_s[0, 0] = 0.0
        v_s[...] = jnp.zeros_like(v_s)

    a = a_ref[...]
    z = a + agg_ref[...]
    b = jnp.maximum(_mlp3(z, w1[...], b1[...], w2[...], b2[...],
                          w3[...], b3[...]), 0.0)
    h = a + rs_ref[0, 0] * b
    g = _mlp3(h, g1[...], gb1[...], g2[...], gb2[...], g3[...], gb3[...])

    m_old = m_s[0, 0]
    m_new = jnp.maximum(m_old, jnp.max(g))
    c = jnp.exp(m_old - m_new)
    w = jnp.exp(g - m_new)
    s_s[0, 0] = s_s[0, 0] * c + jnp.sum(w)
    wv = lax.dot_general(w, h, (((0,), (0,)), ((), ())),
                         preferred_element_type=jnp.float32)
    v_s[...] = v_s[...] * c + wv
    m_s[0, 0] = m_new

    @pl.when(i == pl.num_programs(0) - 1)
    def _():
        o_ref[...] = v_s[...] / s_s[0, 0]


def _final_stage(a, agg, w1, b1, w2, b2, w3, b3, g1, gb1, g2, gb2, g3, gb3, rs):
    grid = (N_NODES // BM,)
    row = pl.BlockSpec((BM, D), lambda i: (i, 0))
    full = lambda x: pl.BlockSpec(x.shape, lambda i: (0,) * x.ndim)
    return pl.pallas_call(
        _final_kernel,
        grid=grid,
        in_specs=[row, row, full(w1), full(b1), full(w2), full(b2),
                  full(w3), full(b3), full(g1), full(gb1), full(g2),
                  full(gb2), full(g3), full(gb3), full(rs)],
        out_specs=pl.BlockSpec((1, D), lambda i: (0, 0)),
        out_shape=jax.ShapeDtypeStruct((1, D), jnp.float32),
        scratch_shapes=[
            pltpu.SMEM((1, 1), jnp.float32),
            pltpu.SMEM((1, 1), jnp.float32),
            pltpu.VMEM((1, D), jnp.float32),
        ],
    )(a, agg, w1, b1, w2, b2, w3, b3, g1, gb1, g2, gb2, g3, gb3, rs)


def _make_sc_segmax(zero_init):
    init_val = 0.0 if zero_init else _NEG_INF
    mesh = plsc.VectorSubcoreMesh(core_axis_name="c", subcore_axis_name="s",
                                  num_cores=2, num_subcores=16)

    @functools.partial(
        pl.kernel,
        out_type=jax.ShapeDtypeStruct((N_PAD, D), jnp.float32),
        mesh=mesh,
        scratch_types=[
            pltpu.VMEM((NPW + 1, D), jnp.float32),
            pltpu.VMEM((CHUNK,), jnp.int32),
            pltpu.VMEM((CHUNK,), jnp.int32),
            pltpu.VMEM((G,), jnp.int32),
            pltpu.VMEM((G + 16,), jnp.int32),
            pltpu.VMEM((G, D), jnp.float32),
            pltpu.SemaphoreType.DMA,
        ],
    )
    def seg(x_hbm, src_hbm, dst_hbm, out_hbm, acc, srcb, dstb, seli, seld,
            msg, sem):
        wid = lax.axis_index("s") * 2 + lax.axis_index("c")
        base = wid * NPW

        def init_row(r, carry):
            for k in range(D // 16):
                acc[r, pl.ds(16 * k, 16)] = jnp.full((16,), init_val,
                                                     jnp.float32)
            return carry
        lax.fori_loop(0, NPW + 1, init_row, 0)
        npw_vec = jnp.full((16,), NPW, jnp.int32)
        slot_iota = lax.iota(jnp.int32, 16)
        for k in range(G // 16):
            seli[pl.ds(16 * k, 16)] = slot_iota + (16 * k)
        for k in range(G // 16 + 1):
            seld[pl.ds(16 * k, 16)] = npw_vec

        def flush(cu):
            seld[pl.ds(cu, 16)] = npw_vec
            seli[pl.ds(cu, 16)] = slot_iota + cu
            pltpu.async_copy(x_hbm.at[seli], msg, sem).wait()

            def upd(jb, carry):
                jo = pl.multiple_of(jb * 16, 8)
                ldv = seld[pl.ds(jo, 16)]
                for jj in range(16):
                    r = ldv[jj]
                    for k in range(D // 16):
                        s = pl.ds(16 * k, 16)
                        acc[r, s] = jnp.maximum(
                            acc[r, s], msg[jb * 16 + jj, s])
                return carry
            lax.fori_loop(0, (cu + 15) >> 4, upd, 0)
            for k in range(G // 16 + 1):
                seld[pl.ds(16 * k, 16)] = npw_vec

        lanes = lax.iota(jnp.int32, 16)
        ones = jnp.full((16,), 1, jnp.int32)
        zeros = jnp.zeros((16,), jnp.int32)

        def chunk_body(c, cursor):
            off = pl.multiple_of(c * CHUNK, 8)
            pltpu.sync_copy(src_hbm.at[pl.ds(off, CHUNK)], srcb)
            pltpu.sync_copy(dst_hbm.at[pl.ds(off, CHUNK)], dstb)

            def sg_body(gsg, cu):
                o = gsg * SG
                lds, svs, prefs, cnts = [], [], [], []
                for k in range(SG // 16):
                    ok = pl.multiple_of(o + 16 * k, 8)
                    ld = dstb[pl.ds(ok, 16)] - base
                    sv = srcb[pl.ds(ok, 16)]
                    m = (ld >= 0) & (ld < NPW)
                    s = jnp.where(m, ones, zeros)
                    for sh in (1, 2, 4, 8):
                        sg_ = s[jnp.maximum(lanes - sh, 0)]
                        s = s + jnp.where(lanes >= sh, sg_, zeros)
                    lds.append(ld)
                    svs.append(sv)
                    prefs.append(s)
                    cnts.append(s[15])

                def do_flush(c0):
                    flush(c0)
                    return 0
                cu = lax.cond(cu > G - SG - 16, do_flush, lambda c0: c0, cu)

                for k in range(SG // 16):
                    def compact(cu, k=k):
                        s = prefs[k]
                        tgt = lanes + 1
                        inv = zeros
                        for step in (8, 4, 2, 1):
                            probe = inv + (step - 1)
                            v = s[jnp.minimum(probe, 15)]
                            inv = jnp.where(v < tgt, inv + step, inv)
                        seli[pl.ds(cu, 16)] = svs[k][inv]
                        seld[pl.ds(cu, 16)] = lds[k][inv]
                        return cu + cnts[k]
                    cu = lax.cond(cnts[k] > 0, compact, lambda c0: c0, cu)
                return cu

            return lax.fori_loop(0, NSG, sg_body, cursor)

        cursor = lax.fori_loop(0, NCHUNK, chunk_body, 0)
        flush(cursor)

        if not zero_init:
            def fix_row(r, carry):
                for k in range(D // 16):
                    s = pl.ds(16 * k, 16)
                    v = acc[r, s]
                    acc[r, s] = jnp.where(v == _NEG_INF, 0.0, v)
                return carry
            lax.fori_loop(0, NPW, fix_row, 0)

        pltpu.sync_copy(acc.at[pl.ds(0, NPW)], out_hbm.at[pl.ds(base, NPW)])

    return seg


_segmax_neg = _make_sc_segmax(zero_init=False)
_segmax_zero = _make_sc_segmax(zero_init=True)


def _segmax(x, src, dst, zero_init):
    fn = _segmax_zero if zero_init else _segmax_neg
    return fn(x, src, dst)[:N_NODES]


def kernel(node_feat, edge_index, W1, b1, W2, b2, W3, b3,
           G1, gb1, G2, gb2, G3, gb3, residual_scale):
    src = edge_index[0].astype(jnp.int32)
    dst = edge_index[1].astype(jnp.int32)
    b1r = b1.reshape(1, HID)
    b2r = b2.reshape(1, HID)
    b3r = b3.reshape(1, D)
    gb1r = gb1.reshape(1, HID)
    gb2r = gb2.reshape(1, HID)
    gb3r = gb3.reshape(1, 1)
    rs = residual_scale.reshape(1, 1)

    agg_x = _segmax(node_feat, src, dst, zero_init=False)
    a = _gin_apply(node_feat, agg_x, W1, b1r, W2, b2r, W3, b3r)
    agg_a = _segmax(a, src, dst, zero_init=True)
    return _final_stage(a, agg_a, W1, b1r, W2, b2r, W3, b3r,
                        G1, gb1r, G2, gb2r, G3, gb3r, rs)


# speedup vs baseline: 5.6904x; 1.1725x over previous
import functools

import jax
import jax.numpy as jnp
from jax import lax
from jax.experimental import pallas as pl
from jax.experimental.pallas import tpu as pltpu
from jax.experimental.pallas import tpu_sc as plsc

N_NODES = 10000
N_EDGES = 160000
D = 256
HID = 64

BM = 1000
_NEG_INF = float("-inf")

NW = 32
NPW = 320
N_PAD = NW * NPW
CHUNK = 3200
SG = 64
NSG = CHUNK // SG
NCHUNK = N_EDGES // CHUNK
G = 128


def _leaky(x):
    return jnp.where(x >= 0, x, 0.01 * x)


def _mlp3(z, w1, b1, w2, b2, w3, b3):
    h = _leaky(jnp.dot(z, w1, preferred_element_type=jnp.float32) + b1)
    h = _leaky(jnp.dot(h, w2, preferred_element_type=jnp.float32) + b2)
    return jnp.dot(h, w3, preferred_element_type=jnp.float32) + b3


def _gin_apply_kernel(x_ref, agg_ref, w1, b1, w2, b2, w3, b3, o_ref):
    z = x_ref[...] + agg_ref[...]
    o = _mlp3(z, w1[...], b1[...], w2[...], b2[...], w3[...], b3[...])
    o_ref[...] = jnp.maximum(o, 0.0)


def _gin_apply(x, agg, w1, b1, w2, b2, w3, b3):
    grid = (N_NODES // BM,)
    row = pl.BlockSpec((BM, D), lambda i: (i, 0))
    full = lambda a: pl.BlockSpec(a.shape, lambda i: (0,) * a.ndim)
    return pl.pallas_call(
        _gin_apply_kernel,
        grid=grid,
        in_specs=[row, row, full(w1), full(b1), full(w2), full(b2),
                  full(w3), full(b3)],
        out_specs=row,
        out_shape=jax.ShapeDtypeStruct((N_NODES, D), jnp.float32),
    )(x, agg, w1, b1, w2, b2, w3, b3)


def _final_kernel(a_ref, agg_ref, w1, b1, w2, b2, w3, b3,
                  g1, gb1, g2, gb2, g3, gb3, rs_ref, o_ref,
                  m_s, s_s, v_s):
    i = pl.program_id(0)

    @pl.when(i == 0)
    def _():
        m_s[0, 0] = _NEG_INF
        s_s[0, 0] = 0.0
        v_s[...] = jnp.zeros_like(v_s)

    a = a_ref[...]
    z = a + agg_ref[...]
    b = jnp.maximum(_mlp3(z, w1[...], b1[...], w2[...], b2[...],
                          w3[...], b3[...]), 0.0)
    h = a + rs_ref[0, 0] * b
    g = _mlp3(h, g1[...], gb1[...], g2[...], gb2[...], g3[...], gb3[...])

    m_old = m_s[0, 0]
    m_new = jnp.maximum(m_old, jnp.max(g))
    c = jnp.exp(m_old - m_new)
    w = jnp.exp(g - m_new)
    s_s[0, 0] = s_s[0, 0] * c + jnp.sum(w)
    wv = lax.dot_general(w, h, (((0,), (0,)), ((), ())),
                         preferred_element_type=jnp.float32)
    v_s[...] = v_s[...] * c + wv
    m_s[0, 0] = m_new

    @pl.when(i == pl.num_programs(0) - 1)
    def _():
        o_ref[...] = v_s[...] / s_s[0, 0]


def _final_stage(a, agg, w1, b1, w2, b2, w3, b3, g1, gb1, g2, gb2, g3, gb3, rs):
    grid = (N_NODES // BM,)
    row = pl.BlockSpec((BM, D), lambda i: (i, 0))
    full = lambda x: pl.BlockSpec(x.shape, lambda i: (0,) * x.ndim)
    return pl.pallas_call(
        _final_kernel,
        grid=grid,
        in_specs=[row, row, full(w1), full(b1), full(w2), full(b2),
                  full(w3), full(b3), full(g1), full(gb1), full(g2),
                  full(gb2), full(g3), full(gb3), full(rs)],
        out_specs=pl.BlockSpec((1, D), lambda i: (0, 0)),
        out_shape=jax.ShapeDtypeStruct((1, D), jnp.float32),
        scratch_shapes=[
            pltpu.SMEM((1, 1), jnp.float32),
            pltpu.SMEM((1, 1), jnp.float32),
            pltpu.VMEM((1, D), jnp.float32),
        ],
    )(a, agg, w1, b1, w2, b2, w3, b3, g1, gb1, g2, gb2, g3, gb3, rs)


def _make_sc_segmax(zero_init):
    init_val = 0.0 if zero_init else _NEG_INF
    mesh = plsc.VectorSubcoreMesh(core_axis_name="c", subcore_axis_name="s",
                                  num_cores=2, num_subcores=16)

    @functools.partial(
        pl.kernel,
        out_type=jax.ShapeDtypeStruct((N_PAD, D), jnp.float32),
        mesh=mesh,
        scratch_types=[
            pltpu.VMEM((NPW + 1, D), jnp.float32),
            pltpu.VMEM((CHUNK,), jnp.int32),
            pltpu.VMEM((CHUNK,), jnp.int32),
            pltpu.VMEM((G,), jnp.int32),
            pltpu.VMEM((G + 16,), jnp.int32),
            pltpu.VMEM((G, D), jnp.float32),
            pltpu.VMEM((G,), jnp.int32),
            pltpu.VMEM((G + 16,), jnp.int32),
            pltpu.SemaphoreType.DMA,
        ],
    )
    def seg(x_hbm, src_hbm, dst_hbm, out_hbm, acc, srcb, dstb, seli, seld,
            msg, sh_seli, sh_seld, sem):
        wid = lax.axis_index("s") * 2 + lax.axis_index("c")
        base = wid * NPW

        def init_row(r, carry):
            for k in range(D // 16):
                acc[r, pl.ds(16 * k, 16)] = jnp.full((16,), init_val,
                                                     jnp.float32)
            return carry
        lax.fori_loop(0, NPW + 1, init_row, 0)
        npw_vec = jnp.full((16,), NPW, jnp.int32)
        slot_iota = lax.iota(jnp.int32, 16)
        for k in range(G // 16):
            seli[pl.ds(16 * k, 16)] = slot_iota + (16 * k)
            sh_seli[pl.ds(16 * k, 16)] = slot_iota + (16 * k)
        for k in range(G // 16 + 1):
            seld[pl.ds(16 * k, 16)] = npw_vec

        def drain(prev):
            pltpu.make_async_copy(x_hbm.at[sh_seli], msg, sem).wait()

            def upd(jb, carry):
                jo = pl.multiple_of(jb * 16, 8)
                ldv = sh_seld[pl.ds(jo, 16)]
                for jj in range(16):
                    r = ldv[jj]
                    for k in range(D // 16):
                        s = pl.ds(16 * k, 16)
                        acc[r, s] = jnp.maximum(
                            acc[r, s], msg[jb * 16 + jj, s])
                return carry
            lax.fori_loop(0, (prev + 15) >> 4, upd, 0)

        pltpu.async_copy(x_hbm.at[sh_seli], msg, sem)

        def flush(cu, prev):
            seld[pl.ds(cu, 16)] = npw_vec
            seli[pl.ds(cu, 16)] = slot_iota + cu
            drain(prev)
            for k in range(G // 16):
                sh_seli[pl.ds(16 * k, 16)] = seli[pl.ds(16 * k, 16)]
            for k in range(G // 16 + 1):
                sh_seld[pl.ds(16 * k, 16)] = seld[pl.ds(16 * k, 16)]
            pltpu.async_copy(x_hbm.at[sh_seli], msg, sem)
            for k in range(G // 16 + 1):
                seld[pl.ds(16 * k, 16)] = npw_vec

        lanes = lax.iota(jnp.int32, 16)
        ones = jnp.full((16,), 1, jnp.int32)
        zeros = jnp.zeros((16,), jnp.int32)

        def chunk_body(c, state):
            off = pl.multiple_of(c * CHUNK, 8)
            pltpu.sync_copy(src_hbm.at[pl.ds(off, CHUNK)], srcb)
            pltpu.sync_copy(dst_hbm.at[pl.ds(off, CHUNK)], dstb)

            def sg_body(gsg, state):
                cu, prev = state
                o = gsg * SG
                lds, svs, prefs, cnts = [], [], [], []
                for k in range(SG // 16):
                    ok = pl.multiple_of(o + 16 * k, 8)
                    ld = dstb[pl.ds(ok, 16)] - base
                    sv = srcb[pl.ds(ok, 16)]
                    m = (ld >= 0) & (ld < NPW)
                    s = jnp.where(m, ones, zeros)
                    for sh in (1, 2, 4, 8):
                        sg_ = s[jnp.maximum(lanes - sh, 0)]
                        s = s + jnp.where(lanes >= sh, sg_, zeros)
                    lds.append(ld)
                    svs.append(sv)
                    prefs.append(s)
                    cnts.append(s[15])

                def do_flush(st):
                    c0, p0 = st
                    flush(c0, p0)
                    return (0, c0)
                cu, prev = lax.cond(cu > G - SG - 16, do_flush,
                                    lambda st: st, (cu, prev))

                for k in range(SG // 16):
                    def compact(cu, k=k):
                        s = prefs[k]
                        tgt = lanes + 1
                        inv = zeros
                        for step in (8, 4, 2, 1):
                            probe = inv + (step - 1)
                            v = s[jnp.minimum(probe, 15)]
                            inv = jnp.where(v < tgt, inv + step, inv)
                        seli[pl.ds(cu, 16)] = svs[k][inv]
                        seld[pl.ds(cu, 16)] = lds[k][inv]
                        return cu + cnts[k]
                    cu = lax.cond(cnts[k] > 0, compact, lambda c0: c0, cu)
                return (cu, prev)

            return lax.fori_loop(0, NSG, sg_body, state)

        cursor, prev = lax.fori_loop(0, NCHUNK, chunk_body, (0, 0))
        flush(cursor, prev)
        drain(cursor)

        if not zero_init:
            def fix_row(r, carry):
                for k in range(D // 16):
                    s = pl.ds(16 * k, 16)
                    v = acc[r, s]
                    acc[r, s] = jnp.where(v == _NEG_INF, 0.0, v)
                return carry
            lax.fori_loop(0, NPW, fix_row, 0)

        pltpu.sync_copy(acc.at[pl.ds(0, NPW)], out_hbm.at[pl.ds(base, NPW)])

    return seg


_segmax_neg = _make_sc_segmax(zero_init=False)
_segmax_zero = _make_sc_segmax(zero_init=True)


def _segmax(x, src, dst, zero_init):
    fn = _segmax_zero if zero_init else _segmax_neg
    return fn(x, src, dst)[:N_NODES]


def kernel(node_feat, edge_index, W1, b1, W2, b2, W3, b3,
           G1, gb1, G2, gb2, G3, gb3, residual_scale):
    src = edge_index[0].astype(jnp.int32)
    dst = edge_index[1].astype(jnp.int32)
    b1r = b1.reshape(1, HID)
    b2r = b2.reshape(1, HID)
    b3r = b3.reshape(1, D)
    gb1r = gb1.reshape(1, HID)
    gb2r = gb2.reshape(1, HID)
    gb3r = gb3.reshape(1, 1)
    rs = residual_scale.reshape(1, 1)

    agg_x = _segmax(node_feat, src, dst, zero_init=False)
    a = _gin_apply(node_feat, agg_x, W1, b1r, W2, b2r, W3, b3r)
    agg_a = _segmax(a, src, dst, zero_init=True)
    return _final_stage(a, agg_a, W1, b1r, W2, b2r, W3, b3r,
                        G1, gb1r, G2, gb2r, G3, gb3r, rs)


# speedup vs baseline: 5.9029x; 1.0374x over previous
import functools

import jax
import jax.numpy as jnp
from jax import lax
from jax.experimental import pallas as pl
from jax.experimental.pallas import tpu as pltpu
from jax.experimental.pallas import tpu_sc as plsc

N_NODES = 10000
N_EDGES = 160000
D = 256
HID = 64

BM = 1000
_NEG_INF = float("-inf")

NW = 32
NPW = 320
N_PAD = NW * NPW
CHUNK = 3200
SG = 64
NSG = CHUNK // SG
NCHUNK = N_EDGES // CHUNK
G = 128


def _leaky(x):
    return jnp.where(x >= 0, x, 0.01 * x)


def _mlp3(z, w1, b1, w2, b2, w3, b3):
    h = _leaky(jnp.dot(z, w1, preferred_element_type=jnp.float32) + b1)
    h = _leaky(jnp.dot(h, w2, preferred_element_type=jnp.float32) + b2)
    return jnp.dot(h, w3, preferred_element_type=jnp.float32) + b3


def _gin_apply_kernel(x_ref, agg_ref, w1, b1, w2, b2, w3, b3, o_ref):
    z = x_ref[...] + agg_ref[...]
    o = _mlp3(z, w1[...], b1[...], w2[...], b2[...], w3[...], b3[...])
    o_ref[...] = jnp.maximum(o, 0.0)


def _gin_apply(x, agg, w1, b1, w2, b2, w3, b3):
    grid = (N_NODES // BM,)
    row = pl.BlockSpec((BM, D), lambda i: (i, 0))
    full = lambda a: pl.BlockSpec(a.shape, lambda i: (0,) * a.ndim)
    return pl.pallas_call(
        _gin_apply_kernel,
        grid=grid,
        in_specs=[row, row, full(w1), full(b1), full(w2), full(b2),
                  full(w3), full(b3)],
        out_specs=row,
        out_shape=jax.ShapeDtypeStruct((N_NODES, D), jnp.float32),
    )(x, agg, w1, b1, w2, b2, w3, b3)


def _final_kernel(a_ref, agg_ref, w1, b1, w2, b2, w3, b3,
                  g1, gb1, g2, gb2, g3, gb3, rs_ref, o_ref,
                  m_s, s_s, v_s):
    i = pl.program_id(0)

    @pl.when(i == 0)
    def _():
        m_s[0, 0] = _NEG_INF
        s_s[0, 0] = 0.0
        v_s[...] = jnp.zeros_like(v_s)

    a = a_ref[...]
    z = a + agg_ref[...]
    b = jnp.maximum(_mlp3(z, w1[...], b1[...], w2[...], b2[...],
                          w3[...], b3[...]), 0.0)
    h = a + rs_ref[0, 0] * b
    g = _mlp3(h, g1[...], gb1[...], g2[...], gb2[...], g3[...], gb3[...])

    m_old = m_s[0, 0]
    m_new = jnp.maximum(m_old, jnp.max(g))
    c = jnp.exp(m_old - m_new)
    w = jnp.exp(g - m_new)
    s_s[0, 0] = s_s[0, 0] * c + jnp.sum(w)
    wv = lax.dot_general(w, h, (((0,), (0,)), ((), ())),
                         preferred_element_type=jnp.float32)
    v_s[...] = v_s[...] * c + wv
    m_s[0, 0] = m_new

    @pl.when(i == pl.num_programs(0) - 1)
    def _():
        o_ref[...] = v_s[...] / s_s[0, 0]


def _final_stage(a, agg, w1, b1, w2, b2, w3, b3, g1, gb1, g2, gb2, g3, gb3, rs):
    grid = (N_NODES // BM,)
    row = pl.BlockSpec((BM, D), lambda i: (i, 0))
    full = lambda x: pl.BlockSpec(x.shape, lambda i: (0,) * x.ndim)
    return pl.pallas_call(
        _final_kernel,
        grid=grid,
        in_specs=[row, row, full(w1), full(b1), full(w2), full(b2),
                  full(w3), full(b3), full(g1), full(gb1), full(g2),
                  full(gb2), full(g3), full(gb3), full(rs)],
        out_specs=pl.BlockSpec((1, D), lambda i: (0, 0)),
        out_shape=jax.ShapeDtypeStruct((1, D), jnp.float32),
        scratch_shapes=[
            pltpu.SMEM((1, 1), jnp.float32),
            pltpu.SMEM((1, 1), jnp.float32),
            pltpu.VMEM((1, D), jnp.float32),
        ],
    )(a, agg, w1, b1, w2, b2, w3, b3, g1, gb1, g2, gb2, g3, gb3, rs)


def _make_sc_segmax(zero_init):
    init_val = 0.0 if zero_init else _NEG_INF
    mesh = plsc.VectorSubcoreMesh(core_axis_name="c", subcore_axis_name="s",
                                  num_cores=2, num_subcores=16)

    @functools.partial(
        pl.kernel,
        out_type=jax.ShapeDtypeStruct((N_PAD, D), jnp.float32),
        mesh=mesh,
        scratch_types=[
            pltpu.VMEM((NPW + 1, D), jnp.float32),
            pltpu.VMEM((CHUNK,), jnp.int32),
            pltpu.VMEM((CHUNK,), jnp.int32),
            pltpu.VMEM((CHUNK,), jnp.int32),
            pltpu.VMEM((CHUNK,), jnp.int32),
            pltpu.VMEM((G,), jnp.int32),
            pltpu.VMEM((G + 16,), jnp.int32),
            pltpu.VMEM((G, D), jnp.float32),
            pltpu.VMEM((G,), jnp.int32),
            pltpu.VMEM((G + 16,), jnp.int32),
            pltpu.SemaphoreType.DMA,
            pltpu.SemaphoreType.DMA,
            pltpu.SemaphoreType.DMA,
        ],
    )
    def seg(x_hbm, src_hbm, dst_hbm, out_hbm, acc, srcb, dstb, srcb2,
            dstb2, seli, seld, msg, sh_seli, sh_seld, sem, semA, semB):
        wid = lax.axis_index("s") * 2 + lax.axis_index("c")
        base = wid * NPW

        def init_row(r, carry):
            for k in range(D // 16):
                acc[r, pl.ds(16 * k, 16)] = jnp.full((16,), init_val,
                                                     jnp.float32)
            return carry
        lax.fori_loop(0, NPW + 1, init_row, 0)
        npw_vec = jnp.full((16,), NPW, jnp.int32)
        slot_iota = lax.iota(jnp.int32, 16)
        for k in range(G // 16):
            seli[pl.ds(16 * k, 16)] = slot_iota + (16 * k)
            sh_seli[pl.ds(16 * k, 16)] = slot_iota + (16 * k)
        for k in range(G // 16 + 1):
            seld[pl.ds(16 * k, 16)] = npw_vec

        def drain(prev):
            pltpu.make_async_copy(x_hbm.at[sh_seli], msg, sem).wait()

            def upd(jb, carry):
                jo = pl.multiple_of(jb * 16, 8)
                ldv = sh_seld[pl.ds(jo, 16)]
                for jj in range(16):
                    r = ldv[jj]
                    for k in range(D // 16):
                        s = pl.ds(16 * k, 16)
                        acc[r, s] = jnp.maximum(
                            acc[r, s], msg[jb * 16 + jj, s])
                return carry
            lax.fori_loop(0, (prev + 15) >> 4, upd, 0)

        pltpu.async_copy(x_hbm.at[sh_seli], msg, sem)

        def flush(cu, prev):
            seld[pl.ds(cu, 16)] = npw_vec
            seli[pl.ds(cu, 16)] = slot_iota + cu
            drain(prev)
            for k in range(G // 16):
                sh_seli[pl.ds(16 * k, 16)] = seli[pl.ds(16 * k, 16)]
            for k in range(G // 16 + 1):
                sh_seld[pl.ds(16 * k, 16)] = seld[pl.ds(16 * k, 16)]
            pltpu.async_copy(x_hbm.at[sh_seli], msg, sem)
            for k in range(G // 16 + 1):
                seld[pl.ds(16 * k, 16)] = npw_vec

        lanes = lax.iota(jnp.int32, 16)
        ones = jnp.full((16,), 1, jnp.int32)
        zeros = jnp.zeros((16,), jnp.int32)

        def issue_chunk(c, sb, db, sm):
            off = pl.multiple_of(c * CHUNK, 8)
            pltpu.async_copy(src_hbm.at[pl.ds(off, CHUNK)], sb, sm)
            pltpu.async_copy(dst_hbm.at[pl.ds(off, CHUNK)], db, sm)

        def wait_chunk(sb, db, sm):
            pltpu.make_async_copy(src_hbm.at[pl.ds(0, CHUNK)], sb,
                                  sm).wait()
            pltpu.make_async_copy(dst_hbm.at[pl.ds(0, CHUNK)], db,
                                  sm).wait()

        def scan_chunk(sb, db, state):
            srcc, dstc = sb, db

            def sg_body(gsg, state):
                cu, prev = state
                o = gsg * SG
                lds, svs, prefs, cnts = [], [], [], []
                for k in range(SG // 16):
                    ok = pl.multiple_of(o + 16 * k, 8)
                    ld = dstc[pl.ds(ok, 16)] - base
                    sv = srcc[pl.ds(ok, 16)]
                    m = (ld >= 0) & (ld < NPW)
                    s = jnp.where(m, ones, zeros)
                    for sh in (1, 2, 4, 8):
                        sg_ = s[jnp.maximum(lanes - sh, 0)]
                        s = s + jnp.where(lanes >= sh, sg_, zeros)
                    lds.append(ld)
                    svs.append(sv)
                    prefs.append(s)
                    cnts.append(s[15])

                def do_flush(st):
                    c0, p0 = st
                    flush(c0, p0)
                    return (0, c0)
                cu, prev = lax.cond(cu > G - SG - 16, do_flush,
                                    lambda st: st, (cu, prev))

                for k in range(SG // 16):
                    def compact(cu, k=k):
                        s = prefs[k]
                        tgt = lanes + 1
                        inv = zeros
                        for step in (8, 4, 2, 1):
                            probe = inv + (step - 1)
                            v = s[jnp.minimum(probe, 15)]
                            inv = jnp.where(v < tgt, inv + step, inv)
                        seli[pl.ds(cu, 16)] = svs[k][inv]
                        seld[pl.ds(cu, 16)] = lds[k][inv]
                        return cu + cnts[k]
                    cu = lax.cond(cnts[k] > 0, compact, lambda c0: c0, cu)
                return (cu, prev)

            return lax.fori_loop(0, NSG, sg_body, state)

        issue_chunk(0, srcb, dstb, semA)

        def pair_body(cp, state):
            c0 = cp * 2
            wait_chunk(srcb, dstb, semA)
            issue_chunk(c0 + 1, srcb2, dstb2, semB)
            state = scan_chunk(srcb, dstb, state)
            wait_chunk(srcb2, dstb2, semB)
            cnext = jnp.where(c0 + 2 < NCHUNK, c0 + 2, 0)
            issue_chunk(cnext, srcb, dstb, semA)
            state = scan_chunk(srcb2, dstb2, state)
            return state

        cursor, prev = lax.fori_loop(0, NCHUNK // 2, pair_body, (0, 0))
        wait_chunk(srcb, dstb, semA)
        flush(cursor, prev)
        drain(cursor)

        if not zero_init:
            def fix_row(r, carry):
                for k in range(D // 16):
                    s = pl.ds(16 * k, 16)
                    v = acc[r, s]
                    acc[r, s] = jnp.where(v == _NEG_INF, 0.0, v)
                return carry
            lax.fori_loop(0, NPW, fix_row, 0)

        pltpu.sync_copy(acc.at[pl.ds(0, NPW)], out_hbm.at[pl.ds(base, NPW)])

    return seg


_segmax_neg = _make_sc_segmax(zero_init=False)
_segmax_zero = _make_sc_segmax(zero_init=True)


def _segmax(x, src, dst, zero_init):
    fn = _segmax_zero if zero_init else _segmax_neg
    return fn(x, src, dst)[:N_NODES]


def kernel(node_feat, edge_index, W1, b1, W2, b2, W3, b3,
           G1, gb1, G2, gb2, G3, gb3, residual_scale):
    src = edge_index[0].astype(jnp.int32)
    dst = edge_index[1].astype(jnp.int32)
    b1r = b1.reshape(1, HID)
    b2r = b2.reshape(1, HID)
    b3r = b3.reshape(1, D)
    gb1r = gb1.reshape(1, HID)
    gb2r = gb2.reshape(1, HID)
    gb3r = gb3.reshape(1, 1)
    rs = residual_scale.reshape(1, 1)

    agg_x = _segmax(node_feat, src, dst, zero_init=False)
    a = _gin_apply(node_feat, agg_x, W1, b1r, W2, b2r, W3, b3r)
    agg_a = _segmax(a, src, dst, zero_init=True)
    return _final_stage(a, agg_a, W1, b1r, W2, b2r, W3, b3r,
                        G1, gb1r, G2, gb2r, G3, gb3r, rs)


# speedup vs baseline: 5.9488x; 1.0078x over previous
import functools

import jax
import jax.numpy as jnp
from jax import lax
from jax.experimental import pallas as pl
from jax.experimental.pallas import tpu as pltpu
from jax.experimental.pallas import tpu_sc as plsc

N_NODES = 10000
N_EDGES = 160000
D = 256
HID = 64

BM = 1000
_NEG_INF = float("-inf")

NW = 32
NPW = 320
N_PAD = NW * NPW
CHUNK = 3200
SG = 64
NSG = CHUNK // SG
NCHUNK = N_EDGES // CHUNK
G = 128


def _leaky(x):
    return jnp.where(x >= 0, x, 0.01 * x)


def _mlp3(z, w1, b1, w2, b2, w3, b3):
    h = _leaky(jnp.dot(z, w1, preferred_element_type=jnp.float32) + b1)
    h = _leaky(jnp.dot(h, w2, preferred_element_type=jnp.float32) + b2)
    return jnp.dot(h, w3, preferred_element_type=jnp.float32) + b3


def _gin_apply_kernel(x_ref, agg_ref, w1, b1, w2, b2, w3, b3, o_ref):
    z = x_ref[...] + agg_ref[...]
    o = _mlp3(z, w1[...], b1[...], w2[...], b2[...], w3[...], b3[...])
    o_ref[...] = jnp.maximum(o, 0.0)


def _gin_apply(x, agg, w1, b1, w2, b2, w3, b3):
    grid = (N_NODES // BM,)
    row = pl.BlockSpec((BM, D), lambda i: (i, 0))
    full = lambda a: pl.BlockSpec(a.shape, lambda i: (0,) * a.ndim)
    return pl.pallas_call(
        _gin_apply_kernel,
        grid=grid,
        in_specs=[row, row, full(w1), full(b1), full(w2), full(b2),
                  full(w3), full(b3)],
        out_specs=row,
        out_shape=jax.ShapeDtypeStruct((N_NODES, D), jnp.float32),
    )(x, agg, w1, b1, w2, b2, w3, b3)


def _final_kernel(a_ref, agg_ref, w1, b1, w2, b2, w3, b3,
                  g1, gb1, g2, gb2, g3, gb3, rs_ref, o_ref,
                  m_s, s_s, v_s):
    i = pl.program_id(0)

    @pl.when(i == 0)
    def _():
        m_s[0, 0] = _NEG_INF
        s_s[0, 0] = 0.0
        v_s[...] = jnp.zeros_like(v_s)

    a = a_ref[...]
    z = a + agg_ref[...]
    b = jnp.maximum(_mlp3(z, w1[...], b1[...], w2[...], b2[...],
                          w3[...], b3[...]), 0.0)
    h = a + rs_ref[0, 0] * b
    g = _mlp3(h, g1[...], gb1[...], g2[...], gb2[...], g3[...], gb3[...])

    m_old = m_s[0, 0]
    m_new = jnp.maximum(m_old, jnp.max(g))
    c = jnp.exp(m_old - m_new)
    w = jnp.exp(g - m_new)
    s_s[0, 0] = s_s[0, 0] * c + jnp.sum(w)
    wv = lax.dot_general(w, h, (((0,), (0,)), ((), ())),
                         preferred_element_type=jnp.float32)
    v_s[...] = v_s[...] * c + wv
    m_s[0, 0] = m_new

    @pl.when(i == pl.num_programs(0) - 1)
    def _():
        o_ref[...] = v_s[...] / s_s[0, 0]


def _final_stage(a, agg, w1, b1, w2, b2, w3, b3, g1, gb1, g2, gb2, g3, gb3, rs):
    grid = (N_NODES // BM,)
    row = pl.BlockSpec((BM, D), lambda i: (i, 0))
    full = lambda x: pl.BlockSpec(x.shape, lambda i: (0,) * x.ndim)
    return pl.pallas_call(
        _final_kernel,
        grid=grid,
        in_specs=[row, row, full(w1), full(b1), full(w2), full(b2),
                  full(w3), full(b3), full(g1), full(gb1), full(g2),
                  full(gb2), full(g3), full(gb3), full(rs)],
        out_specs=pl.BlockSpec((1, D), lambda i: (0, 0)),
        out_shape=jax.ShapeDtypeStruct((1, D), jnp.float32),
        scratch_shapes=[
            pltpu.SMEM((1, 1), jnp.float32),
            pltpu.SMEM((1, 1), jnp.float32),
            pltpu.VMEM((1, D), jnp.float32),
        ],
    )(a, agg, w1, b1, w2, b2, w3, b3, g1, gb1, g2, gb2, g3, gb3, rs)


def _make_sc_segmax(zero_init):
    init_val = 0.0 if zero_init else _NEG_INF
    mesh = plsc.VectorSubcoreMesh(core_axis_name="c", subcore_axis_name="s",
                                  num_cores=2, num_subcores=16)

    @functools.partial(
        pl.kernel,
        out_type=jax.ShapeDtypeStruct((N_PAD, D), jnp.float32),
        mesh=mesh,
        scratch_types=[
            pltpu.VMEM((NPW + 1, D), jnp.float32),
            pltpu.VMEM((CHUNK,), jnp.int32),
            pltpu.VMEM((CHUNK,), jnp.int32),
            pltpu.VMEM((CHUNK,), jnp.int32),
            pltpu.VMEM((CHUNK,), jnp.int32),
            pltpu.VMEM((G,), jnp.int32),
            pltpu.VMEM((G + 16,), jnp.int32),
            pltpu.VMEM((G, D), jnp.float32),
            pltpu.VMEM((G,), jnp.int32),
            pltpu.VMEM((G + 16,), jnp.int32),
            pltpu.SemaphoreType.DMA,
            pltpu.SemaphoreType.DMA,
            pltpu.SemaphoreType.DMA,
        ],
    )
    def seg(x_hbm, src_hbm, dst_hbm, out_hbm, acc, srcb, dstb, srcb2,
            dstb2, seli, seld, msg, sh_seli, sh_seld, sem, semA, semB):
        wid = lax.axis_index("s") * 2 + lax.axis_index("c")
        base = wid * NPW

        def init_row(r, carry):
            for k in range(D // 16):
                acc[r, pl.ds(16 * k, 16)] = jnp.full((16,), init_val,
                                                     jnp.float32)
            return carry
        lax.fori_loop(0, NPW + 1, init_row, 0)
        npw_vec = jnp.full((16,), NPW, jnp.int32)
        slot_iota = lax.iota(jnp.int32, 16)
        for k in range(G // 16):
            seli[pl.ds(16 * k, 16)] = slot_iota + (16 * k)
            sh_seli[pl.ds(16 * k, 16)] = slot_iota + (16 * k)
        for k in range(G // 16 + 1):
            seld[pl.ds(16 * k, 16)] = npw_vec

        def drain(prev):
            pltpu.make_async_copy(x_hbm.at[sh_seli], msg, sem).wait()

            def upd(jb, carry):
                jo = pl.multiple_of(jb * 16, 8)
                ldv = sh_seld[pl.ds(jo, 16)]
                for jj in range(16):
                    r = ldv[jj]
                    for k in range(D // 16):
                        s = pl.ds(16 * k, 16)
                        acc[r, s] = jnp.maximum(
                            acc[r, s], msg[jb * 16 + jj, s])
                return carry
            lax.fori_loop(0, (prev + 15) >> 4, upd, 0)

        pltpu.async_copy(x_hbm.at[sh_seli], msg, sem)

        def flush(cu, prev):
            seld[pl.ds(cu, 16)] = npw_vec
            seli[pl.ds(cu, 16)] = slot_iota + cu
            drain(prev)
            for k in range(G // 16):
                sh_seli[pl.ds(16 * k, 16)] = seli[pl.ds(16 * k, 16)]
            for k in range(G // 16 + 1):
                sh_seld[pl.ds(16 * k, 16)] = seld[pl.ds(16 * k, 16)]
            pltpu.async_copy(x_hbm.at[sh_seli], msg, sem)
            for k in range(G // 16 + 1):
                seld[pl.ds(16 * k, 16)] = npw_vec

        lanes = lax.iota(jnp.int32, 16)
        ones = jnp.full((16,), 1, jnp.int32)
        zeros = jnp.zeros((16,), jnp.int32)

        def issue_chunk(c, sb, db, sm):
            off = pl.multiple_of(c * CHUNK, 8)
            pltpu.async_copy(src_hbm.at[pl.ds(off, CHUNK)], sb, sm)
            pltpu.async_copy(dst_hbm.at[pl.ds(off, CHUNK)], db, sm)

        def wait_chunk(sb, db, sm):
            pltpu.make_async_copy(src_hbm.at[pl.ds(0, CHUNK)], sb,
                                  sm).wait()
            pltpu.make_async_copy(dst_hbm.at[pl.ds(0, CHUNK)], db,
                                  sm).wait()

        def scan_chunk(sb, db, state):
            srcc, dstc = sb, db

            def sg_body(gsg, state):
                cu, prev = state
                o = gsg * SG
                oks, dvs, prefs, cnts = [], [], [], []
                for k in range(SG // 16):
                    ok = pl.multiple_of(o + 16 * k, 8)
                    dv = dstc[pl.ds(ok, 16)]
                    m = (dv >= base) & (dv < base + NPW)
                    s = jnp.where(m, ones, zeros)
                    for sh in (1, 2, 4, 8):
                        sg_ = s[jnp.maximum(lanes - sh, 0)]
                        s = s + jnp.where(lanes >= sh, sg_, zeros)
                    oks.append(ok)
                    dvs.append(dv)
                    prefs.append(s)
                    cnts.append(s[15])

                def do_flush(st):
                    c0, p0 = st
                    flush(c0, p0)
                    return (0, c0)
                cu, prev = lax.cond(cu > G - SG - 16, do_flush,
                                    lambda st: st, (cu, prev))

                for k in range(SG // 16):
                    def compact(cu, k=k):
                        s = prefs[k]
                        tgt = lanes + 1
                        inv = zeros
                        for step in (8, 4, 2, 1):
                            probe = inv + (step - 1)
                            v = s[jnp.minimum(probe, 15)]
                            inv = jnp.where(v < tgt, inv + step, inv)
                        sv = srcc[pl.ds(oks[k], 16)]
                        seli[pl.ds(cu, 16)] = sv[inv]
                        seld[pl.ds(cu, 16)] = dvs[k][inv] - base
                        return cu + cnts[k]
                    cu = lax.cond(cnts[k] > 0, compact, lambda c0: c0, cu)
                return (cu, prev)

            return lax.fori_loop(0, NSG, sg_body, state)

        issue_chunk(0, srcb, dstb, semA)

        def pair_body(cp, state):
            c0 = cp * 2
            wait_chunk(srcb, dstb, semA)
            issue_chunk(c0 + 1, srcb2, dstb2, semB)
            state = scan_chunk(srcb, dstb, state)
            wait_chunk(srcb2, dstb2, semB)
            cnext = jnp.where(c0 + 2 < NCHUNK, c0 + 2, 0)
            issue_chunk(cnext, srcb, dstb, semA)
            state = scan_chunk(srcb2, dstb2, state)
            return state

        cursor, prev = lax.fori_loop(0, NCHUNK // 2, pair_body, (0, 0))
        wait_chunk(srcb, dstb, semA)
        flush(cursor, prev)
        drain(cursor)

        if not zero_init:
            def fix_row(r, carry):
                for k in range(D // 16):
                    s = pl.ds(16 * k, 16)
                    v = acc[r, s]
                    acc[r, s] = jnp.where(v == _NEG_INF, 0.0, v)
                return carry
            lax.fori_loop(0, NPW, fix_row, 0)

        pltpu.sync_copy(acc.at[pl.ds(0, NPW)], out_hbm.at[pl.ds(base, NPW)])

    return seg


_segmax_neg = _make_sc_segmax(zero_init=False)
_segmax_zero = _make_sc_segmax(zero_init=True)


def _segmax(x, src, dst, zero_init):
    fn = _segmax_zero if zero_init else _segmax_neg
    return fn(x, src, dst)[:N_NODES]


def kernel(node_feat, edge_index, W1, b1, W2, b2, W3, b3,
           G1, gb1, G2, gb2, G3, gb3, residual_scale):
    src = edge_index[0].astype(jnp.int32)
    dst = edge_index[1].astype(jnp.int32)
    b1r = b1.reshape(1, HID)
    b2r = b2.reshape(1, HID)
    b3r = b3.reshape(1, D)
    gb1r = gb1.reshape(1, HID)
    gb2r = gb2.reshape(1, HID)
    gb3r = gb3.reshape(1, 1)
    rs = residual_scale.reshape(1, 1)

    agg_x = _segmax(node_feat, src, dst, zero_init=False)
    a = _gin_apply(node_feat, agg_x, W1, b1r, W2, b2r, W3, b3r)
    agg_a = _segmax(a, src, dst, zero_init=True)
    return _final_stage(a, agg_a, W1, b1r, W2, b2r, W3, b3r,
                        G1, gb1r, G2, gb2r, G3, gb3r, rs)
